# baseline (speedup 1.0000x reference)
import functools

import jax
import jax.numpy as jnp
from jax import lax
from jax.experimental import pallas as pl
from jax.experimental.pallas import tpu as pltpu
from jax.experimental.pallas import tpu_sc as plsc

NC = 2
NS = 16
NW = NC * NS
L = 16
CH = 80


def _mesh():
    return plsc.VectorSubcoreMesh(core_axis_name="c", subcore_axis_name="s")


def _widx(cid, sid):
    return sid * NC + cid


@functools.cache
def _make_gnn_scatter(npad, d, e):
    npart = npad // NW

    @functools.partial(
        pl.kernel,
        out_type=jax.ShapeDtypeStruct((npad, d), jnp.float32),
        mesh=_mesh(),
        scratch_types=[
            pltpu.VMEM((CH,), jnp.int32),
            pltpu.VMEM((CH,), jnp.int32),
            pltpu.VMEM((CH,), jnp.int32),
            pltpu.VMEM((CH,), jnp.float32),
            pltpu.VMEM((L,), jnp.int32),
            pltpu.VMEM((CH, 128), jnp.float32),
            pltpu.VMEM((320, 128), jnp.float32),
            pltpu.SemaphoreType.DMA,
        ],
    )
    def k(h_hbm, src_hbm, dst_hbm, w_hbm, lists_hbm, cnts_hbm, z_hbm,
          out_hbm, eids, srcv, didx, wv, cbuf, rows, acc, sem):
        cid = lax.axis_index("c")
        sid = lax.axis_index("s")
        wid = _widx(cid, sid)
        lane = lax.broadcasted_iota(jnp.int32, (L,), 0)

        def zrow(r, c0):
            for dd in range(d // L):
                acc[r, pl.ds(dd * L, L)] = jnp.zeros((L,), jnp.float32)
            return c0
        lax.fori_loop(0, npart, zrow, 0)

        pltpu.sync_copy(cnts_hbm.at[pl.ds(wid * 8, 8)], cbuf.at[pl.ds(0, 8)])
        mycnt = cbuf[pl.ds(0, L)][0]
        trips = (mycnt + CH - 1) // CH

        def body(t, carry):
            pltpu.sync_copy(lists_hbm.at[pl.ds(wid * e + t * CH, CH)], eids)
            pltpu.async_copy(src_hbm.at[eids], srcv, sem).wait()
            pltpu.async_copy(dst_hbm.at[eids], didx, sem).wait()
            pltpu.async_copy(w_hbm.at[eids], wv, sem).wait()
            pltpu.async_copy(h_hbm.at[srcv], rows, sem).wait()
            base = wid * npart
            for g in range(CH // L):
                valid = (t * CH + g * L + lane) < mycnt
                wvec = jnp.where(valid, wv[pl.ds(g * L, L)], 0.0)
                rl = didx[pl.ds(g * L, L)] - base
                rl = jnp.minimum(jnp.maximum(rl, 0), npart - 1)
                for i in range(L):
                    ws = jnp.broadcast_to(wvec[i], (L,))
                    r = rl[i]
                    for dd in range(d // L):
                        cs = pl.ds(dd * L, L)
                        acc[r, cs] = acc[r, cs] + rows[g * L + i, cs] * ws
            return carry
        lax.fori_loop(0, trips, body, 0)

        pltpu.sync_copy(acc, out_hbm.at[pl.ds(wid * npart, npart)])

    return k


@functools.cache
def _make_sw_scatter(npad, e):
    npart = npad // NW

    @functools.partial(
        pl.kernel,
        out_type=jax.ShapeDtypeStruct((npad, 128), jnp.float32),
        mesh=_mesh(),
        scratch_types=[
            pltpu.VMEM((CH,), jnp.int32),
            pltpu.VMEM((CH,), jnp.int32),
            pltpu.VMEM((CH,), jnp.float32),
            pltpu.VMEM((L,), jnp.int32),
            pltpu.VMEM((CH, 128), jnp.float32),
            pltpu.VMEM((320, 128), jnp.float32),
            pltpu.SemaphoreType.DMA,
        ],
    )
    def k(dst_hbm, w_hbm, lists_hbm, cnts_hbm, z_hbm, out_hbm,
          eids, didx, wv, cbuf, wrows, accw, sem):
        cid = lax.axis_index("c")
        sid = lax.axis_index("s")
        wid = _widx(cid, sid)
        lane = lax.broadcasted_iota(jnp.int32, (L,), 0)

        def zrow(r, c0):
            for dd in range(128 // L):
                accw[r, pl.ds(dd * L, L)] = jnp.zeros((L,), jnp.float32)
            return c0
        lax.fori_loop(0, npart, zrow, 0)

        pltpu.sync_copy(cnts_hbm.at[pl.ds(wid * 8, 8)], cbuf.at[pl.ds(0, 8)])
        mycnt = cbuf[pl.ds(0, L)][0]
        trips = (mycnt + CH - 1) // CH

        def body(t, carry):
            pltpu.sync_copy(lists_hbm.at[pl.ds(wid * e + t * CH, CH)], eids)
            pltpu.async_copy(dst_hbm.at[eids], didx, sem).wait()
            pltpu.async_copy(w_hbm.at[eids], wv, sem).wait()
            base = wid * npart
            for g in range(CH // L):
                valid = (t * CH + g * L + lane) < mycnt
                wvec = jnp.where(valid, wv[pl.ds(g * L, L)], 0.0)
                rl = didx[pl.ds(g * L, L)] - base
                rl = jnp.minimum(jnp.maximum(rl, 0), npart - 1)
                for i in range(L):
                    ws = jnp.broadcast_to(wvec[i], (L,))
                    r = rl[i]
                    cs = pl.ds(0, L)
                    accw[r, cs] = accw[r, cs] + ws
            return carry
        lax.fori_loop(0, trips, body, 0)

        pltpu.sync_copy(accw, out_hbm.at[pl.ds(wid * npart, npart)])

    return k


@functools.cache
def _make_gnn_dense(n, d, b):
    def body(agg_ref, sw_ref, h_ref, wn_ref, wr_ref, br_ref, g_ref,
             bb_ref, o_ref):
        sw = jnp.maximum(sw_ref[...][:, 0:1], 1.0)
        out = agg_ref[...] / sw
        h2 = (jnp.dot(out, wn_ref[...], preferred_element_type=jnp.float32)
              + jnp.dot(h_ref[...], wr_ref[...],
                        preferred_element_type=jnp.float32)
              + br_ref[...])
        r = jnp.maximum(h2, 0.0)
        m = jnp.mean(r, axis=-1, keepdims=True)
        cdev = r - m
        v = jnp.mean(cdev * cdev, axis=-1, keepdims=True)
        o_ref[...] = cdev * lax.rsqrt(v + 1e-5) * g_ref[...] + bb_ref[...]

    return pl.pallas_call(
        body,
        grid=(n // b,),
        in_specs=[
            pl.BlockSpec((b, d), lambda i: (i, 0)),
            pl.BlockSpec((b, 128), lambda i: (i, 0)),
            pl.BlockSpec((b, d), lambda i: (i, 0)),
            pl.BlockSpec((d, d), lambda i: (0, 0)),
            pl.BlockSpec((d, d), lambda i: (0, 0)),
            pl.BlockSpec((1, d), lambda i: (0, 0)),
            pl.BlockSpec((1, d), lambda i: (0, 0)),
            pl.BlockSpec((1, d), lambda i: (0, 0)),
        ],
        out_specs=pl.BlockSpec((b, d), lambda i: (i, 0)),
        out_shape=jax.ShapeDtypeStruct((n, d), jnp.float32),
    )


@functools.cache
def _make_mlp_first(n, din, dout, b):
    def body(h_ref, w_ref, b_ref, z_ref, s_ref, q_ref):
        z = (jnp.dot(h_ref[...], w_ref[...],
                     preferred_element_type=jnp.float32) + b_ref[...])
        z_ref[...] = z

        @pl.when(pl.program_id(0) == 0)
        def _():
            s_ref[...] = jnp.zeros_like(s_ref)
            q_ref[...] = jnp.zeros_like(q_ref)
        s_ref[...] += jnp.sum(z, axis=0, keepdims=True)
        q_ref[...] += jnp.sum(z * z, axis=0, keepdims=True)

    return pl.pallas_call(
        body,
        grid=(n // b,),
        in_specs=[
            pl.BlockSpec((b, din), lambda i: (i, 0)),
            pl.BlockSpec((din, dout), lambda i: (0, 0)),
            pl.BlockSpec((1, dout), lambda i: (0, 0)),
        ],
        out_specs=[
            pl.BlockSpec((b, dout), lambda i: (i, 0)),
            pl.BlockSpec((1, dout), lambda i: (0, 0)),
            pl.BlockSpec((1, dout), lambda i: (0, 0)),
        ],
        out_shape=[
            jax.ShapeDtypeStruct((n, dout), jnp.float32),
            jax.ShapeDtypeStruct((1, dout), jnp.float32),
            jax.ShapeDtypeStruct((1, dout), jnp.float32),
        ],
    )


@functools.cache
def _make_mlp_mid(n, din, dout, b):
    def body(z_ref, s_ref, q_ref, g_ref, bb_ref, w_ref, b2_ref,
             o_ref, s2_ref, q2_ref):
        m = s_ref[...] / n
        v = q_ref[...] / n - m * m
        a = jnp.maximum((z_ref[...] - m) * lax.rsqrt(v + 1e-5) * g_ref[...]
                        + bb_ref[...], 0.0)
        z2 = (jnp.dot(a, w_ref[...], preferred_element_type=jnp.float32)
              + b2_ref[...])
        o_ref[...] = z2

        @pl.when(pl.program_id(0) == 0)
        def _():
            s2_ref[...] = jnp.zeros_like(s2_ref)
            q2_ref[...] = jnp.zeros_like(q2_ref)
        s2_ref[...] += jnp.sum(z2, axis=0, keepdims=True)
        q2_ref[...] += jnp.sum(z2 * z2, axis=0, keepdims=True)

    return pl.pallas_call(
        body,
        grid=(n // b,),
        in_specs=[
            pl.BlockSpec((b, din), lambda i: (i, 0)),
            pl.BlockSpec((1, din), lambda i: (0, 0)),
            pl.BlockSpec((1, din), lambda i: (0, 0)),
            pl.BlockSpec((1, din), lambda i: (0, 0)),
            pl.BlockSpec((1, din), lambda i: (0, 0)),
            pl.BlockSpec((din, dout), lambda i: (0, 0)),
            pl.BlockSpec((1, dout), lambda i: (0, 0)),
        ],
        out_specs=[
            pl.BlockSpec((b, dout), lambda i: (i, 0)),
            pl.BlockSpec((1, dout), lambda i: (0, 0)),
            pl.BlockSpec((1, dout), lambda i: (0, 0)),
        ],
        out_shape=[
            jax.ShapeDtypeStruct((n, dout), jnp.float32),
            jax.ShapeDtypeStruct((1, dout), jnp.float32),
            jax.ShapeDtypeStruct((1, dout), jnp.float32),
        ],
    )


@functools.cache
def _make_mlp_last(n, din, b):
    def body(z_ref, s_ref, q_ref, g_ref, bb_ref, w_ref, b2_ref, o_ref):
        m = s_ref[...] / n
        v = q_ref[...] / n - m * m
        a = jnp.maximum((z_ref[...] - m) * lax.rsqrt(v + 1e-5) * g_ref[...]
                        + bb_ref[...], 0.0)
        o_ref[...] = (jnp.dot(a, w_ref[...],
                              preferred_element_type=jnp.float32)
                      + b2_ref[...])

    return pl.pallas_call(
        body,
        grid=(n // b,),
        in_specs=[
            pl.BlockSpec((b, din), lambda i: (i, 0)),
            pl.BlockSpec((1, din), lambda i: (0, 0)),
            pl.BlockSpec((1, din), lambda i: (0, 0)),
            pl.BlockSpec((1, din), lambda i: (0, 0)),
            pl.BlockSpec((1, din), lambda i: (0, 0)),
            pl.BlockSpec((din, 1), lambda i: (0, 0)),
            pl.BlockSpec((1, 1), lambda i: (0, 0)),
        ],
        out_specs=pl.BlockSpec((b, 1), lambda i: (i, 0)),
        out_shape=jax.ShapeDtypeStruct((n, 1), jnp.float32),
    )


def kernel(x, edge_index, edge_weight, params):
    n, d = x.shape
    e = edge_weight.shape[0]
    b = n // 10
    src = edge_index[0]
    dst = edge_index[1]
    npad = ((n + NW * 8 - 1) // (NW * 8)) * (NW * 8)
    npart = npad // NW

    owner = dst // npart
    order = jnp.argsort(owner)
    owner_sorted = owner[order]
    cnts = jnp.bincount(owner, length=NW)
    offs = jnp.concatenate([jnp.zeros((1,), cnts.dtype),
                            jnp.cumsum(cnts)[:-1]])
    posn = owner_sorted * e + (jnp.arange(e) - offs[owner_sorted])
    lists = jnp.zeros((NW * e,), jnp.int32).at[posn].set(
        order.astype(jnp.int32))
    cnts8 = jnp.zeros((NW * 8,), jnp.int32).at[jnp.arange(NW) * 8].set(
        cnts.astype(jnp.int32))

    zrows = jnp.zeros((CH, 128), jnp.float32)
    sww = _make_sw_scatter(npad, e)(dst, edge_weight, lists, cnts8, zrows)
    scat = _make_gnn_scatter(npad, d, e)
    dense = _make_gnn_dense(npad, d, npad // 10)
    h = jnp.pad(x, ((0, npad - n), (0, 0)))
    for p in params['gnn']:
        agg = scat(h, src, dst, edge_weight, lists, cnts8, zrows)
        h = dense(agg, sww, h, p['Wn'].T, p['Wr'].T, p['br'][None, :],
                  p['ln_g'][None, :], p['ln_b'][None, :])
    h = h[:n]

    c = params['clf']
    z1, s1, q1 = _make_mlp_first(n, d, 256, b)(h, c['W1'].T, c['b1'][None, :])
    z2, s2, q2 = _make_mlp_mid(n, 256, 128, b)(
        z1, s1, q1, c['g1'][None, :], c['bb1'][None, :], c['W2'].T,
        c['b2'][None, :])
    z3, s3, q3 = _make_mlp_mid(n, 128, 64, b)(
        z2, s2, q2, c['g2'][None, :], c['bb2'][None, :], c['W3'].T,
        c['b3'][None, :])
    lg = _make_mlp_last(n, 64, b)(
        z3, s3, q3, c['g3'][None, :], c['bb3'][None, :], c['W4'].T,
        c['b4'][None, :])
    return lg[:, 0]


# speedup vs baseline: 1.0859x; 1.0859x over previous
import functools

import jax
import jax.numpy as jnp
from jax import lax
from jax.experimental import pallas as pl
from jax.experimental.pallas import tpu as pltpu
from jax.experimental.pallas import tpu_sc as plsc

NC = 2
NS = 16
NW = NC * NS
L = 16
CH = 128


def _mesh():
    return plsc.VectorSubcoreMesh(core_axis_name="c", subcore_axis_name="s")


def _widx(cid, sid):
    return sid * NC + cid


@functools.cache
def _make_gnn_scatter(npad, d, e):
    npart = npad // NW

    @functools.partial(
        pl.kernel,
        out_type=jax.ShapeDtypeStruct((npad, d), jnp.float32),
        mesh=_mesh(),
        scratch_types=[
            pltpu.VMEM((CH,), jnp.int32),
            pltpu.VMEM((CH,), jnp.int32),
            pltpu.VMEM((CH,), jnp.int32),
            pltpu.VMEM((CH,), jnp.float32),
            pltpu.VMEM((L,), jnp.int32),
            pltpu.VMEM((CH, 128), jnp.float32),
            pltpu.VMEM((320, 128), jnp.float32),
            pltpu.SemaphoreType.DMA,
        ],
    )
    def k(h_hbm, src_hbm, dst_hbm, w_hbm, lists_hbm, cnts_hbm, z_hbm,
          out_hbm, eids, srcv, didx, wv, cbuf, rows, acc, sem):
        cid = lax.axis_index("c")
        sid = lax.axis_index("s")
        wid = _widx(cid, sid)
        lane = lax.broadcasted_iota(jnp.int32, (L,), 0)

        def zrow(r, c0):
            for dd in range(d // L):
                acc[r, pl.ds(dd * L, L)] = jnp.zeros((L,), jnp.float32)
            return c0
        lax.fori_loop(0, npart, zrow, 0)

        pltpu.sync_copy(cnts_hbm.at[pl.ds(wid * 8, 8)], cbuf.at[pl.ds(0, 8)])
        mycnt = cbuf[pl.ds(0, L)][0]
        trips = (mycnt + CH - 1) // CH

        def body(t, carry):
            pltpu.sync_copy(lists_hbm.at[pl.ds(wid * e + t * CH, CH)], eids)
            c1 = pltpu.async_copy(src_hbm.at[eids], srcv, sem)
            c2 = pltpu.async_copy(dst_hbm.at[eids], didx, sem)
            c3 = pltpu.async_copy(w_hbm.at[eids], wv, sem)
            c1.wait()
            c2.wait()
            c3.wait()
            pltpu.async_copy(h_hbm.at[srcv], rows, sem).wait()
            base = wid * npart
            for g in range(CH // L):
                valid = (t * CH + g * L + lane) < mycnt
                wvec = jnp.where(valid, wv[pl.ds(g * L, L)], 0.0)
                rl = didx[pl.ds(g * L, L)] - base
                rl = jnp.minimum(jnp.maximum(rl, 0), npart - 1)
                for i in range(L):
                    ws = jnp.broadcast_to(wvec[i], (L,))
                    r = rl[i]
                    for dd in range(d // L):
                        cs = pl.ds(dd * L, L)
                        acc[r, cs] = acc[r, cs] + rows[g * L + i, cs] * ws
            return carry
        lax.fori_loop(0, trips, body, 0)

        pltpu.sync_copy(acc, out_hbm.at[pl.ds(wid * npart, npart)])

    return k


@functools.cache
def _make_sw_scatter(npad, e):
    npart = npad // NW

    @functools.partial(
        pl.kernel,
        out_type=jax.ShapeDtypeStruct((npad, 128), jnp.float32),
        mesh=_mesh(),
        scratch_types=[
            pltpu.VMEM((CH,), jnp.int32),
            pltpu.VMEM((CH,), jnp.int32),
            pltpu.VMEM((CH,), jnp.float32),
            pltpu.VMEM((L,), jnp.int32),
            pltpu.VMEM((CH, 128), jnp.float32),
            pltpu.VMEM((320, 128), jnp.float32),
            pltpu.SemaphoreType.DMA,
        ],
    )
    def k(dst_hbm, w_hbm, lists_hbm, cnts_hbm, z_hbm, out_hbm,
          eids, didx, wv, cbuf, wrows, accw, sem):
        cid = lax.axis_index("c")
        sid = lax.axis_index("s")
        wid = _widx(cid, sid)
        lane = lax.broadcasted_iota(jnp.int32, (L,), 0)

        def zrow(r, c0):
            for dd in range(128 // L):
                accw[r, pl.ds(dd * L, L)] = jnp.zeros((L,), jnp.float32)
            return c0
        lax.fori_loop(0, npart, zrow, 0)

        pltpu.sync_copy(cnts_hbm.at[pl.ds(wid * 8, 8)], cbuf.at[pl.ds(0, 8)])
        mycnt = cbuf[pl.ds(0, L)][0]
        trips = (mycnt + CH - 1) // CH

        def body(t, carry):
            pltpu.sync_copy(lists_hbm.at[pl.ds(wid * e + t * CH, CH)], eids)
            c1 = pltpu.async_copy(dst_hbm.at[eids], didx, sem)
            c2 = pltpu.async_copy(w_hbm.at[eids], wv, sem)
            c1.wait()
            c2.wait()
            base = wid * npart
            for g in range(CH // L):
                valid = (t * CH + g * L + lane) < mycnt
                wvec = jnp.where(valid, wv[pl.ds(g * L, L)], 0.0)
                rl = didx[pl.ds(g * L, L)] - base
                rl = jnp.minimum(jnp.maximum(rl, 0), npart - 1)
                for i in range(L):
                    ws = jnp.broadcast_to(wvec[i], (L,))
                    r = rl[i]
                    cs = pl.ds(0, L)
                    accw[r, cs] = accw[r, cs] + ws
            return carry
        lax.fori_loop(0, trips, body, 0)

        pltpu.sync_copy(accw, out_hbm.at[pl.ds(wid * npart, npart)])

    return k


@functools.cache
def _make_gnn_dense(n, d, b):
    def body(agg_ref, sw_ref, h_ref, wn_ref, wr_ref, br_ref, g_ref,
             bb_ref, o_ref):
        sw = jnp.maximum(sw_ref[...][:, 0:1], 1.0)
        out = agg_ref[...] / sw
        h2 = (jnp.dot(out, wn_ref[...], preferred_element_type=jnp.float32)
              + jnp.dot(h_ref[...], wr_ref[...],
                        preferred_element_type=jnp.float32)
              + br_ref[...])
        r = jnp.maximum(h2, 0.0)
        m = jnp.mean(r, axis=-1, keepdims=True)
        cdev = r - m
        v = jnp.mean(cdev * cdev, axis=-1, keepdims=True)
        o_ref[...] = cdev * lax.rsqrt(v + 1e-5) * g_ref[...] + bb_ref[...]

    return pl.pallas_call(
        body,
        grid=(n // b,),
        in_specs=[
            pl.BlockSpec((b, d), lambda i: (i, 0)),
            pl.BlockSpec((b, 128), lambda i: (i, 0)),
            pl.BlockSpec((b, d), lambda i: (i, 0)),
            pl.BlockSpec((d, d), lambda i: (0, 0)),
            pl.BlockSpec((d, d), lambda i: (0, 0)),
            pl.BlockSpec((1, d), lambda i: (0, 0)),
            pl.BlockSpec((1, d), lambda i: (0, 0)),
            pl.BlockSpec((1, d), lambda i: (0, 0)),
        ],
        out_specs=pl.BlockSpec((b, d), lambda i: (i, 0)),
        out_shape=jax.ShapeDtypeStruct((n, d), jnp.float32),
    )


@functools.cache
def _make_mlp_first(n, din, dout, b):
    def body(h_ref, w_ref, b_ref, z_ref, s_ref, q_ref):
        z = (jnp.dot(h_ref[...], w_ref[...],
                     preferred_element_type=jnp.float32) + b_ref[...])
        z_ref[...] = z

        @pl.when(pl.program_id(0) == 0)
        def _():
            s_ref[...] = jnp.zeros_like(s_ref)
            q_ref[...] = jnp.zeros_like(q_ref)
        s_ref[...] += jnp.sum(z, axis=0, keepdims=True)
        q_ref[...] += jnp.sum(z * z, axis=0, keepdims=True)

    return pl.pallas_call(
        body,
        grid=(n // b,),
        in_specs=[
            pl.BlockSpec((b, din), lambda i: (i, 0)),
            pl.BlockSpec((din, dout), lambda i: (0, 0)),
            pl.BlockSpec((1, dout), lambda i: (0, 0)),
        ],
        out_specs=[
            pl.BlockSpec((b, dout), lambda i: (i, 0)),
            pl.BlockSpec((1, dout), lambda i: (0, 0)),
            pl.BlockSpec((1, dout), lambda i: (0, 0)),
        ],
        out_shape=[
            jax.ShapeDtypeStruct((n, dout), jnp.float32),
            jax.ShapeDtypeStruct((1, dout), jnp.float32),
            jax.ShapeDtypeStruct((1, dout), jnp.float32),
        ],
    )


@functools.cache
def _make_mlp_mid(n, din, dout, b):
    def body(z_ref, s_ref, q_ref, g_ref, bb_ref, w_ref, b2_ref,
             o_ref, s2_ref, q2_ref):
        m = s_ref[...] / n
        v = q_ref[...] / n - m * m
        a = jnp.maximum((z_ref[...] - m) * lax.rsqrt(v + 1e-5) * g_ref[...]
                        + bb_ref[...], 0.0)
        z2 = (jnp.dot(a, w_ref[...], preferred_element_type=jnp.float32)
              + b2_ref[...])
        o_ref[...] = z2

        @pl.when(pl.program_id(0) == 0)
        def _():
            s2_ref[...] = jnp.zeros_like(s2_ref)
            q2_ref[...] = jnp.zeros_like(q2_ref)
        s2_ref[...] += jnp.sum(z2, axis=0, keepdims=True)
        q2_ref[...] += jnp.sum(z2 * z2, axis=0, keepdims=True)

    return pl.pallas_call(
        body,
        grid=(n // b,),
        in_specs=[
            pl.BlockSpec((b, din), lambda i: (i, 0)),
            pl.BlockSpec((1, din), lambda i: (0, 0)),
            pl.BlockSpec((1, din), lambda i: (0, 0)),
            pl.BlockSpec((1, din), lambda i: (0, 0)),
            pl.BlockSpec((1, din), lambda i: (0, 0)),
            pl.BlockSpec((din, dout), lambda i: (0, 0)),
            pl.BlockSpec((1, dout), lambda i: (0, 0)),
        ],
        out_specs=[
            pl.BlockSpec((b, dout), lambda i: (i, 0)),
            pl.BlockSpec((1, dout), lambda i: (0, 0)),
            pl.BlockSpec((1, dout), lambda i: (0, 0)),
        ],
        out_shape=[
            jax.ShapeDtypeStruct((n, dout), jnp.float32),
            jax.ShapeDtypeStruct((1, dout), jnp.float32),
            jax.ShapeDtypeStruct((1, dout), jnp.float32),
        ],
    )


@functools.cache
def _make_mlp_last(n, din, b):
    def body(z_ref, s_ref, q_ref, g_ref, bb_ref, w_ref, b2_ref, o_ref):
        m = s_ref[...] / n
        v = q_ref[...] / n - m * m
        a = jnp.maximum((z_ref[...] - m) * lax.rsqrt(v + 1e-5) * g_ref[...]
                        + bb_ref[...], 0.0)
        o_ref[...] = (jnp.dot(a, w_ref[...],
                              preferred_element_type=jnp.float32)
                      + b2_ref[...])

    return pl.pallas_call(
        body,
        grid=(n // b,),
        in_specs=[
            pl.BlockSpec((b, din), lambda i: (i, 0)),
            pl.BlockSpec((1, din), lambda i: (0, 0)),
            pl.BlockSpec((1, din), lambda i: (0, 0)),
            pl.BlockSpec((1, din), lambda i: (0, 0)),
            pl.BlockSpec((1, din), lambda i: (0, 0)),
            pl.BlockSpec((din, 1), lambda i: (0, 0)),
            pl.BlockSpec((1, 1), lambda i: (0, 0)),
        ],
        out_specs=pl.BlockSpec((b, 1), lambda i: (i, 0)),
        out_shape=jax.ShapeDtypeStruct((n, 1), jnp.float32),
    )


def kernel(x, edge_index, edge_weight, params):
    n, d = x.shape
    e = edge_weight.shape[0]
    b = n // 10
    src = edge_index[0]
    dst = edge_index[1]
    npad = ((n + NW * 8 - 1) // (NW * 8)) * (NW * 8)
    npart = npad // NW

    owner = dst // npart
    order = jnp.argsort(owner)
    owner_sorted = owner[order]
    cnts = jnp.bincount(owner, length=NW)
    offs = jnp.concatenate([jnp.zeros((1,), cnts.dtype),
                            jnp.cumsum(cnts)[:-1]])
    posn = owner_sorted * e + (jnp.arange(e) - offs[owner_sorted])
    lists = jnp.zeros((NW * e,), jnp.int32).at[posn].set(
        order.astype(jnp.int32))
    cnts8 = jnp.zeros((NW * 8,), jnp.int32).at[jnp.arange(NW) * 8].set(
        cnts.astype(jnp.int32))

    zrows = jnp.zeros((CH, 128), jnp.float32)
    sww = _make_sw_scatter(npad, e)(dst, edge_weight, lists, cnts8, zrows)
    scat = _make_gnn_scatter(npad, d, e)
    dense = _make_gnn_dense(npad, d, npad // 10)
    h = jnp.pad(x, ((0, npad - n), (0, 0)))
    for p in params['gnn']:
        agg = scat(h, src, dst, edge_weight, lists, cnts8, zrows)
        h = dense(agg, sww, h, p['Wn'].T, p['Wr'].T, p['br'][None, :],
                  p['ln_g'][None, :], p['ln_b'][None, :])
    h = h[:n]

    c = params['clf']
    z1, s1, q1 = _make_mlp_first(n, d, 256, b)(h, c['W1'].T, c['b1'][None, :])
    z2, s2, q2 = _make_mlp_mid(n, 256, 128, b)(
        z1, s1, q1, c['g1'][None, :], c['bb1'][None, :], c['W2'].T,
        c['b2'][None, :])
    z3, s3, q3 = _make_mlp_mid(n, 128, 64, b)(
        z2, s2, q2, c['g2'][None, :], c['bb2'][None, :], c['W3'].T,
        c['b3'][None, :])
    lg = _make_mlp_last(n, 64, b)(
        z3, s3, q3, c['g3'][None, :], c['bb3'][None, :], c['W4'].T,
        c['b4'][None, :])
    return lg[:, 0]
